# baseline (speedup 1.0000x reference)
import functools

import jax
import jax.numpy as jnp
from jax import lax
from jax.experimental import pallas as pl
from jax.experimental.pallas import tpu as pltpu
from jax.experimental.pallas import tpu_sc as plsc

VOCAB = 100000
EMB_DIM = 64
HIDDEN = 256
LABELS = 100
BATCH = 16384
SEQ = 200

NUM_WORKERS = 32
CHUNK_ROWS = 4
GATHER_W = 100
LANES = 16
LANE_CHUNKS = EMB_DIM // LANES
GSTEPS = 8
GROUP_ROWS = GSTEPS * CHUNK_ROWS


def _pooled_sums(x, emb):
    nb = x.shape[0]
    rows_per_worker = nb // NUM_WORKERS
    steps = rows_per_worker // CHUNK_ROWS
    groups = steps // GSTEPS
    mesh = plsc.VectorSubcoreMesh(core_axis_name="c", subcore_axis_name="s")

    @functools.partial(
        pl.kernel,
        out_type=jax.ShapeDtypeStruct((nb, EMB_DIM), jnp.float32),
        mesh=mesh,
        scratch_types=[
            pltpu.VMEM((2, GROUP_ROWS, SEQ), jnp.int32),
            pltpu.VMEM((2, CHUNK_ROWS * SEQ, EMB_DIM), jnp.float32),
            pltpu.VMEM((GROUP_ROWS, EMB_DIM), jnp.float32),
            pltpu.SemaphoreType.DMA,
            pltpu.SemaphoreType.DMA,
            pltpu.SemaphoreType.DMA,
            pltpu.SemaphoreType.DMA,
        ],
        compiler_params=pltpu.CompilerParams(use_tc_tiling_on_sc=False),
    )
    def k(x_hbm, emb_hbm, out_hbm, idx_v, rows_v, stage_v,
          sem_r0, sem_r1, sem_i0, sem_i1):
        core = lax.axis_index("c")
        sub = lax.axis_index("s")
        wid = sub * 2 + core
        sem_r = [sem_r0, sem_r1]
        sem_i = [sem_i0, sem_i1]

        def idx_src(g):
            return x_hbm.at[pl.ds(wid * rows_per_worker + g * GROUP_ROWS,
                                  GROUP_ROWS)]

        PARTS = ((0, 104), (104, 96))

        def gchunks(ib, k_, rb):
            for r in range(CHUNK_ROWS):
                for off, size in PARTS:
                    yield (
                        idx_v.at[ib, k_ * CHUNK_ROWS + r, pl.ds(off, size)],
                        rows_v.at[rb, pl.ds(r * SEQ + off, size)],
                    )

        def fire(ib, k_, rb):
            for isrc, rdst in gchunks(ib, k_, rb):
                pltpu.async_copy(emb_hbm.at[isrc], rdst, sem_r[rb])

        def drain(ib, k_, rb):
            for isrc, rdst in gchunks(ib, k_, rb):
                pltpu.make_async_copy(emb_hbm.at[isrc], rdst,
                                      sem_r[rb]).wait()

        def reduce_store(k_, rb):
            @pl.loop(0, CHUNK_ROWS)
            def _(r):
                def body(t, accs):
                    return tuple(
                        accs[ci] + rows_v[rb, r * SEQ + t,
                                          pl.ds(ci * LANES, LANES)]
                        for ci in range(LANE_CHUNKS)
                    )
                accs = lax.fori_loop(
                    0, SEQ, body,
                    tuple(jnp.zeros((LANES,), jnp.float32)
                          for _ in range(LANE_CHUNKS)),
                    unroll=8,
                )
                for ci in range(LANE_CHUNKS):
                    stage_v[k_ * CHUNK_ROWS + r,
                            pl.ds(ci * LANES, LANES)] = accs[ci]

        def group_body(G, ib):
            @pl.when(G + 1 < groups)
            def _():
                pltpu.async_copy(idx_src(G + 1), idx_v.at[ib ^ 1],
                                 sem_i[ib ^ 1])

            @pl.loop(0, GSTEPS, step=2)
            def _(k_):
                fire(ib, k_ + 1, 1)
                drain(ib, k_, 0)
                reduce_store(k_, 0)

                @pl.when(k_ + 2 < GSTEPS)
                def _():
                    fire(ib, k_ + 2, 0)

                @pl.when(jnp.logical_and(k_ + 2 == GSTEPS, G + 1 < groups))
                def _():
                    pltpu.make_async_copy(idx_src(G + 1), idx_v.at[ib ^ 1],
                                          sem_i[ib ^ 1]).wait()
                    fire(ib ^ 1, 0, 0)

                drain(ib, k_ + 1, 1)
                reduce_store(k_ + 1, 1)

            pltpu.sync_copy(
                stage_v,
                out_hbm.at[pl.ds(wid * rows_per_worker + G * GROUP_ROWS,
                                 GROUP_ROWS)])

        pltpu.sync_copy(idx_src(0), idx_v.at[0])
        fire(0, 0, 0)

        @pl.loop(0, groups, step=2)
        def _(G):
            group_body(G, 0)
            group_body(G + 1, 1)

    return k(x, emb)


def _mlp_head(pooled, W1, b1, W2, b2):
    nb = pooled.shape[0]
    BB = 2048

    def body(h_ref, w1_ref, b1_ref, w2_ref, b2_ref, out_ref, pred_ref):
        h = h_ref[...] / jnp.float32(SEQ)
        z = jnp.dot(h.astype(jnp.bfloat16), w1_ref[...].astype(jnp.bfloat16),
                    preferred_element_type=jnp.float32) + b1_ref[...]
        z = jnp.maximum(z, 0.0)
        y = jnp.dot(z.astype(jnp.bfloat16), w2_ref[...].astype(jnp.bfloat16),
                    preferred_element_type=jnp.float32) + b2_ref[...]
        out_ref[...] = y
        m = jnp.max(y, axis=-1, keepdims=True)
        iota = lax.broadcasted_iota(jnp.int32, y.shape, 1)
        idx = jnp.where(y == m, iota, jnp.int32(LABELS))
        pred_ref[...] = jnp.min(idx, axis=-1, keepdims=True)

    out, pred = pl.pallas_call(
        body,
        grid=(nb // BB,),
        in_specs=[
            pl.BlockSpec((BB, EMB_DIM), lambda i: (i, 0)),
            pl.BlockSpec((EMB_DIM, HIDDEN), lambda i: (0, 0)),
            pl.BlockSpec((1, HIDDEN), lambda i: (0, 0)),
            pl.BlockSpec((HIDDEN, LABELS), lambda i: (0, 0)),
            pl.BlockSpec((1, LABELS), lambda i: (0, 0)),
        ],
        out_specs=[
            pl.BlockSpec((BB, LABELS), lambda i: (i, 0)),
            pl.BlockSpec((BB, 1), lambda i: (i, 0)),
        ],
        out_shape=[
            jax.ShapeDtypeStruct((nb, LABELS), jnp.float32),
            jax.ShapeDtypeStruct((nb, 1), jnp.int32),
        ],
    )(pooled, W1, b1.reshape(1, HIDDEN), W2, b2.reshape(1, LABELS))
    return out, pred.reshape(nb)


def kernel(x, emb, W1, b1, W2, b2):
    half = BATCH // 2
    outs = []
    for xc in (x[:half], x[half:]):
        pooled = _pooled_sums(xc, emb)
        outs.append(_mlp_head(pooled, W1, b1, W2, b2))
    return (jnp.concatenate([o for o, _ in outs], axis=0),
            jnp.concatenate([p for _, p in outs], axis=0))


# speedup vs baseline: 1.0339x; 1.0339x over previous
import functools

import jax
import jax.numpy as jnp
from jax import lax
from jax.experimental import pallas as pl
from jax.experimental.pallas import tpu as pltpu
from jax.experimental.pallas import tpu_sc as plsc

VOCAB = 100000
EMB_DIM = 64
HIDDEN = 256
LABELS = 100
BATCH = 16384
SEQ = 200

NUM_WORKERS = 32
CHUNK_ROWS = 4
GATHER_W = 100
LANES = 16
LANE_CHUNKS = EMB_DIM // LANES
GSTEPS = 8
GROUP_ROWS = GSTEPS * CHUNK_ROWS


def _pooled_sums(x, emb):
    nb = x.shape[0]
    rows_per_worker = nb // NUM_WORKERS
    steps = rows_per_worker // CHUNK_ROWS
    groups = steps // GSTEPS
    mesh = plsc.VectorSubcoreMesh(core_axis_name="c", subcore_axis_name="s")

    @functools.partial(
        pl.kernel,
        out_type=jax.ShapeDtypeStruct((nb, EMB_DIM), jnp.float32),
        mesh=mesh,
        scratch_types=[
            pltpu.VMEM((2, GROUP_ROWS, SEQ), jnp.int32),
            pltpu.VMEM((2, CHUNK_ROWS * SEQ, EMB_DIM), jnp.float32),
            pltpu.VMEM((GROUP_ROWS, EMB_DIM), jnp.float32),
            pltpu.SemaphoreType.DMA,
            pltpu.SemaphoreType.DMA,
            pltpu.SemaphoreType.DMA,
            pltpu.SemaphoreType.DMA,
        ],
        compiler_params=pltpu.CompilerParams(use_tc_tiling_on_sc=False),
    )
    def k(x_hbm, emb_hbm, out_hbm, idx_v, rows_v, stage_v,
          sem_r0, sem_r1, sem_i0, sem_i1):
        core = lax.axis_index("c")
        sub = lax.axis_index("s")
        wid = sub * 2 + core
        sem_r = [sem_r0, sem_r1]
        sem_i = [sem_i0, sem_i1]

        def idx_src(g):
            return x_hbm.at[pl.ds(wid * rows_per_worker + g * GROUP_ROWS,
                                  GROUP_ROWS)]

        PARTS = ((0, 104), (104, 96))

        def gchunks(ib, k_, rb):
            for r in range(CHUNK_ROWS):
                for off, size in PARTS:
                    yield (
                        idx_v.at[ib, k_ * CHUNK_ROWS + r, pl.ds(off, size)],
                        rows_v.at[rb, pl.ds(r * SEQ + off, size)],
                    )

        def fire(ib, k_, rb):
            for isrc, rdst in gchunks(ib, k_, rb):
                pltpu.async_copy(emb_hbm.at[isrc], rdst, sem_r[rb])

        def drain(ib, k_, rb):
            for isrc, rdst in gchunks(ib, k_, rb):
                pltpu.make_async_copy(emb_hbm.at[isrc], rdst,
                                      sem_r[rb]).wait()

        def reduce_store(k_, rb):
            @pl.loop(0, CHUNK_ROWS)
            def _(r):
                def body(t, accs):
                    return tuple(
                        accs[ci] + rows_v[rb, r * SEQ + t,
                                          pl.ds(ci * LANES, LANES)]
                        for ci in range(LANE_CHUNKS)
                    )
                accs = lax.fori_loop(
                    0, SEQ, body,
                    tuple(jnp.zeros((LANES,), jnp.float32)
                          for _ in range(LANE_CHUNKS)),
                    unroll=8,
                )
                for ci in range(LANE_CHUNKS):
                    stage_v[k_ * CHUNK_ROWS + r,
                            pl.ds(ci * LANES, LANES)] = accs[ci]

        def group_body(G, ib):
            @pl.when(G + 1 < groups)
            def _():
                pltpu.async_copy(idx_src(G + 1), idx_v.at[ib ^ 1],
                                 sem_i[ib ^ 1])

            @pl.loop(0, GSTEPS, step=2)
            def _(k_):
                fire(ib, k_ + 1, 1)
                drain(ib, k_, 0)
                reduce_store(k_, 0)

                @pl.when(k_ + 2 < GSTEPS)
                def _():
                    fire(ib, k_ + 2, 0)

                @pl.when(jnp.logical_and(k_ + 2 == GSTEPS, G + 1 < groups))
                def _():
                    pltpu.make_async_copy(idx_src(G + 1), idx_v.at[ib ^ 1],
                                          sem_i[ib ^ 1]).wait()
                    fire(ib ^ 1, 0, 0)

                drain(ib, k_ + 1, 1)
                reduce_store(k_ + 1, 1)

            pltpu.sync_copy(
                stage_v,
                out_hbm.at[pl.ds(wid * rows_per_worker + G * GROUP_ROWS,
                                 GROUP_ROWS)])

        pltpu.sync_copy(idx_src(0), idx_v.at[0])
        fire(0, 0, 0)

        @pl.loop(0, groups, step=2)
        def _(G):
            group_body(G, 0)
            group_body(G + 1, 1)

    return k(x, emb)


def _mlp_head(pooled, W1, b1, W2, b2):
    nb = pooled.shape[0]
    BB = 2048

    def body(h_ref, w1_ref, b1_ref, w2_ref, b2_ref, out_ref, pred_ref):
        h = h_ref[...] / jnp.float32(SEQ)
        z = jnp.dot(h.astype(jnp.bfloat16), w1_ref[...].astype(jnp.bfloat16),
                    preferred_element_type=jnp.float32) + b1_ref[...]
        z = jnp.maximum(z, 0.0)
        y = jnp.dot(z.astype(jnp.bfloat16), w2_ref[...].astype(jnp.bfloat16),
                    preferred_element_type=jnp.float32) + b2_ref[...]
        out_ref[...] = y
        m = jnp.max(y, axis=-1, keepdims=True)
        iota = lax.broadcasted_iota(jnp.int32, y.shape, 1)
        idx = jnp.where(y == m, iota, jnp.int32(LABELS))
        pred_ref[...] = jnp.min(idx, axis=-1)

    out, pred = pl.pallas_call(
        body,
        grid=(nb // BB,),
        in_specs=[
            pl.BlockSpec((BB, EMB_DIM), lambda i: (i, 0)),
            pl.BlockSpec((EMB_DIM, HIDDEN), lambda i: (0, 0)),
            pl.BlockSpec((1, HIDDEN), lambda i: (0, 0)),
            pl.BlockSpec((HIDDEN, LABELS), lambda i: (0, 0)),
            pl.BlockSpec((1, LABELS), lambda i: (0, 0)),
        ],
        out_specs=[
            pl.BlockSpec((BB, LABELS), lambda i: (i, 0)),
            pl.BlockSpec((BB,), lambda i: (i,)),
        ],
        out_shape=[
            jax.ShapeDtypeStruct((nb, LABELS), jnp.float32),
            jax.ShapeDtypeStruct((nb,), jnp.int32),
        ],
    )(pooled, W1, b1.reshape(1, HIDDEN), W2, b2.reshape(1, LABELS))
    return out, pred


def kernel(x, emb, W1, b1, W2, b2):
    pooled = _pooled_sums(x, emb)
    return _mlp_head(pooled, W1, b1, W2, b2)


# speedup vs baseline: 1.0407x; 1.0066x over previous
import functools

import jax
import jax.numpy as jnp
from jax import lax
from jax.experimental import pallas as pl
from jax.experimental.pallas import tpu as pltpu
from jax.experimental.pallas import tpu_sc as plsc

VOCAB = 100000
EMB_DIM = 64
HIDDEN = 256
LABELS = 100
BATCH = 16384
SEQ = 200

NUM_WORKERS = 32
CHUNK_ROWS = 4
GATHER_W = 100
LANES = 16
LANE_CHUNKS = EMB_DIM // LANES
GSTEPS = 8
GROUP_ROWS = GSTEPS * CHUNK_ROWS


def _pooled_sums(x, emb):
    nb = x.shape[0]
    rows_per_worker = nb // NUM_WORKERS
    steps = rows_per_worker // CHUNK_ROWS
    groups = steps // GSTEPS
    mesh = plsc.VectorSubcoreMesh(core_axis_name="c", subcore_axis_name="s")

    @functools.partial(
        pl.kernel,
        out_type=jax.ShapeDtypeStruct((nb, EMB_DIM), jnp.float32),
        mesh=mesh,
        scratch_types=[
            pltpu.VMEM((2, GROUP_ROWS, SEQ), jnp.int32),
            pltpu.VMEM((2, CHUNK_ROWS * SEQ, EMB_DIM), jnp.float32),
            pltpu.VMEM((GROUP_ROWS, EMB_DIM), jnp.float32),
            pltpu.SemaphoreType.DMA,
            pltpu.SemaphoreType.DMA,
            pltpu.SemaphoreType.DMA,
            pltpu.SemaphoreType.DMA,
        ],
        compiler_params=pltpu.CompilerParams(use_tc_tiling_on_sc=False),
    )
    def k(x_hbm, emb_hbm, out_hbm, idx_v, rows_v, stage_v,
          sem_r0, sem_r1, sem_i0, sem_i1):
        core = lax.axis_index("c")
        sub = lax.axis_index("s")
        wid = sub * 2 + core
        sem_r = [sem_r0, sem_r1]
        sem_i = [sem_i0, sem_i1]

        def idx_src(g):
            return x_hbm.at[pl.ds(wid * rows_per_worker + g * GROUP_ROWS,
                                  GROUP_ROWS)]

        PARTS = ((0, 104), (104, 96))

        def gchunks(ib, k_, rb):
            for r in range(CHUNK_ROWS):
                for off, size in PARTS:
                    yield (
                        idx_v.at[ib, k_ * CHUNK_ROWS + r, pl.ds(off, size)],
                        rows_v.at[rb, pl.ds(r * SEQ + off, size)],
                    )

        def fire(ib, k_, rb):
            for isrc, rdst in gchunks(ib, k_, rb):
                pltpu.async_copy(emb_hbm.at[isrc], rdst, sem_r[rb])

        def drain(ib, k_, rb):
            for isrc, rdst in gchunks(ib, k_, rb):
                pltpu.make_async_copy(emb_hbm.at[isrc], rdst,
                                      sem_r[rb]).wait()

        def reduce_store(k_, rb):
            @pl.loop(0, CHUNK_ROWS)
            def _(r):
                def body(t, accs):
                    return tuple(
                        accs[ci] + rows_v[rb, r * SEQ + t,
                                          pl.ds(ci * LANES, LANES)]
                        for ci in range(LANE_CHUNKS)
                    )
                accs = lax.fori_loop(
                    0, SEQ, body,
                    tuple(jnp.zeros((LANES,), jnp.float32)
                          for _ in range(LANE_CHUNKS)),
                    unroll=8,
                )
                for ci in range(LANE_CHUNKS):
                    stage_v[k_ * CHUNK_ROWS + r,
                            pl.ds(ci * LANES, LANES)] = accs[ci]

        def group_body(G, ib):
            @pl.when(G + 1 < groups)
            def _():
                pltpu.async_copy(idx_src(G + 1), idx_v.at[ib ^ 1],
                                 sem_i[ib ^ 1])

            @pl.loop(0, GSTEPS, step=2)
            def _(k_):
                fire(ib, k_ + 1, 1)
                drain(ib, k_, 0)
                reduce_store(k_, 0)

                @pl.when(k_ + 2 < GSTEPS)
                def _():
                    fire(ib, k_ + 2, 0)

                @pl.when(jnp.logical_and(k_ + 2 == GSTEPS, G + 1 < groups))
                def _():
                    pltpu.make_async_copy(idx_src(G + 1), idx_v.at[ib ^ 1],
                                          sem_i[ib ^ 1]).wait()
                    fire(ib ^ 1, 0, 0)

                drain(ib, k_ + 1, 1)
                reduce_store(k_ + 1, 1)

            pltpu.sync_copy(
                stage_v,
                out_hbm.at[pl.ds(wid * rows_per_worker + G * GROUP_ROWS,
                                 GROUP_ROWS)])

        pltpu.sync_copy(idx_src(0), idx_v.at[0])
        fire(0, 0, 0)

        @pl.loop(0, groups, step=2)
        def _(G):
            group_body(G, 0)
            group_body(G + 1, 1)

    return k(x, emb)


def _mlp_head(pooled, W1, b1, W2, b2):
    nb = pooled.shape[0]
    BB = 2048

    def body(h_ref, w1_ref, b1_ref, w2_ref, b2_ref, out_ref, pred_ref):
        h = h_ref[...] / jnp.float32(SEQ)
        z = jnp.dot(h.astype(jnp.bfloat16), w1_ref[...].astype(jnp.bfloat16),
                    preferred_element_type=jnp.float32) + b1_ref[...]
        z = jnp.maximum(z, 0.0)
        y = jnp.dot(z.astype(jnp.bfloat16), w2_ref[...].astype(jnp.bfloat16),
                    preferred_element_type=jnp.float32) + b2_ref[...]
        out_ref[...] = y
        m = jnp.max(y, axis=-1, keepdims=True)
        iota = lax.broadcasted_iota(jnp.int32, y.shape, 1)
        idx = jnp.where(y == m, iota, jnp.int32(LABELS))
        pred_ref[...] = jnp.min(idx, axis=-1, keepdims=True)

    out, pred = pl.pallas_call(
        body,
        grid=(nb // BB,),
        in_specs=[
            pl.BlockSpec((BB, EMB_DIM), lambda i: (i, 0)),
            pl.BlockSpec((EMB_DIM, HIDDEN), lambda i: (0, 0)),
            pl.BlockSpec((1, HIDDEN), lambda i: (0, 0)),
            pl.BlockSpec((HIDDEN, LABELS), lambda i: (0, 0)),
            pl.BlockSpec((1, LABELS), lambda i: (0, 0)),
        ],
        out_specs=[
            pl.BlockSpec((BB, LABELS), lambda i: (i, 0)),
            pl.BlockSpec((BB, 1), lambda i: (i, 0)),
        ],
        out_shape=[
            jax.ShapeDtypeStruct((nb, LABELS), jnp.float32),
            jax.ShapeDtypeStruct((nb, 1), jnp.int32),
        ],
    )(pooled, W1, b1.reshape(1, HIDDEN), W2, b2.reshape(1, LABELS))
    return out, pred.reshape(nb)


def kernel(x, emb, W1, b1, W2, b2):
    pooled = _pooled_sums(x, emb)
    return _mlp_head(pooled, W1, b1, W2, b2)
